# Initial kernel scaffold; baseline (speedup 1.0000x reference)
import jax
import jax.numpy as jnp
from jax.experimental import pallas as pl


def kernel(s_seq, z_seq, edge_index, W):
    raise NotImplementedError("write your pallas kernel here")


# speedup vs baseline: 27.8512x; 27.8512x over previous
import functools

import jax
import jax.numpy as jnp
from jax import lax
from jax.experimental import pallas as pl
from jax.experimental.pallas import tpu as pltpu
from jax.experimental.pallas import tpu_sc as plsc

N = 10000
C = 256
T = 4
E = 160000
CW = 128
NCH = (T * C) // CW
EB = 125
NTILES = 16
NCORES = 2
NPAD = 10240
ROWS_PER_TILE = NPAD // NTILES
NB = 1000
EPS = 1e-12

_sc_mesh = functools.partial(
    plsc.VectorSubcoreMesh, core_axis_name="c", subcore_axis_name="s")


def _deg_body(dst_hbm, ones_hbm, zeros_hbm, out_hbm, ones_v, zeros_v, idx_v, acc_sh, sem):
    cidx = lax.axis_index("c")
    sidx = lax.axis_index("s")
    pltpu.sync_copy(ones_hbm, ones_v)
    pltpu.sync_copy(zeros_hbm, zeros_v)
    pltpu.sync_copy(dst_hbm.at[cidx, sidx], idx_v)
    for k in range(ROWS_PER_TILE // 32):
        pltpu.sync_copy(zeros_v, acc_sh.at[pl.ds(sidx * ROWS_PER_TILE + k * 32, 32)])
    plsc.subcore_barrier()

    def body(j, carry):
        pltpu.sync_copy(ones_v, acc_sh.at[idx_v.at[j]], add=True)
        return carry

    lax.fori_loop(0, E // (NCORES * NTILES * EB), body, 0)
    plsc.subcore_barrier()
    pltpu.sync_copy(acc_sh.at[pl.ds(sidx * ROWS_PER_TILE, ROWS_PER_TILE)],
                    out_hbm.at[cidx, pl.ds(sidx * ROWS_PER_TILE, ROWS_PER_TILE)])


def _make_deg_kernel():
    return pl.kernel(
        _deg_body,
        mesh=_sc_mesh(),
        out_type=jax.ShapeDtypeStruct((NCORES, NPAD, CW), jnp.float32),
        scratch_types=[
            pltpu.VMEM((EB, CW), jnp.float32),
            pltpu.VMEM((32, CW), jnp.float32),
            pltpu.VMEM((E // (NCORES * NTILES * EB), EB), jnp.int32),
            pltpu.VMEM_SHARED((NPAD, CW), jnp.float32),
            pltpu.SemaphoreType.DMA,
        ],
    )


def _mm_scale_body(s_ref, part_ref, w_ref, q8_ref, deg_ref):
    deg = part_ref[0, :, 0:1] + part_ref[1, :, 0:1] + 1.0
    deg_ref[...] = deg
    dinv = lax.rsqrt(jnp.maximum(deg, EPS))
    w = w_ref[...]
    for t in range(T):
        xwt = lax.dot_general(s_ref[t], w, (((1,), (0,)), ((), ())),
                              preferred_element_type=jnp.float32)
        qt = xwt * dinv
        q8_ref[2 * t] = qt[:, :CW]
        q8_ref[2 * t + 1] = qt[:, CW:]


def _mm_scale_call(s_seq, part, W):
    return pl.pallas_call(
        _mm_scale_body,
        grid=(N // NB,),
        in_specs=[
            pl.BlockSpec((T, NB, C), lambda nb: (0, nb, 0)),
            pl.BlockSpec((NCORES, NB, CW), lambda nb: (0, nb, 0)),
            pl.BlockSpec((C, C), lambda nb: (0, 0)),
        ],
        out_specs=[
            pl.BlockSpec((NCH, NB, CW), lambda nb: (0, nb, 0)),
            pl.BlockSpec((NB, 1), lambda nb: (nb, 0)),
        ],
        out_shape=[
            jax.ShapeDtypeStruct((NCH, N, CW), jnp.float32),
            jax.ShapeDtypeStruct((N, 1), jnp.float32),
        ],
    )(s_seq, part, W)


def _agg_body(p_hbm, srcoff_hbm, dst_hbm, zeros_hbm, out_hbm,
              sidx_v, dst_v, rows_v, zeros_v, acc_sh, sem):
    cidx = lax.axis_index("c")
    sidx = lax.axis_index("s")
    nbatch = E // (NTILES * EB)
    pltpu.sync_copy(zeros_hbm, zeros_v)
    pltpu.sync_copy(dst_hbm.at[sidx], dst_v)
    for cc in range(NCH // NCORES):
        chunk = cidx * (NCH // NCORES) + cc
        pltpu.sync_copy(srcoff_hbm.at[chunk, sidx], sidx_v)
        for k in range(ROWS_PER_TILE // 32):
            pltpu.sync_copy(
                zeros_v, acc_sh.at[pl.ds(sidx * ROWS_PER_TILE + k * 32, 32)])
        plsc.subcore_barrier()

        def body(j, carry):
            pltpu.async_copy(p_hbm.at[sidx_v.at[j]], rows_v, sem).wait()
            pltpu.sync_copy(rows_v, acc_sh.at[dst_v.at[j]], add=True)
            return carry

        lax.fori_loop(0, nbatch, body, 0)
        plsc.subcore_barrier()
        pltpu.sync_copy(
            acc_sh.at[pl.ds(sidx * ROWS_PER_TILE, ROWS_PER_TILE)],
            out_hbm.at[chunk, pl.ds(sidx * ROWS_PER_TILE, ROWS_PER_TILE)])


def _make_agg_kernel():
    nbatch = E // (NTILES * EB)
    return pl.kernel(
        _agg_body,
        mesh=_sc_mesh(),
        out_type=jax.ShapeDtypeStruct((NCH, NPAD, CW), jnp.float32),
        scratch_types=[
            pltpu.VMEM((nbatch, EB), jnp.int32),
            pltpu.VMEM((nbatch, EB), jnp.int32),
            pltpu.VMEM((EB, CW), jnp.float32),
            pltpu.VMEM((32, CW), jnp.float32),
            pltpu.VMEM_SHARED((NPAD, CW), jnp.float32),
            pltpu.SemaphoreType.DMA,
        ],
    )


def _final_body(agg_ref, q_ref, deg_ref, z_ref, o_ref, znew_ref):
    dinv = lax.rsqrt(jnp.maximum(deg_ref[...], EPS))
    xs = []
    for t in range(T):
        aggt = jnp.concatenate([agg_ref[2 * t], agg_ref[2 * t + 1]], axis=1)
        qt = jnp.concatenate([q_ref[2 * t], q_ref[2 * t + 1]], axis=1)
        xs.append((aggt + qt) * dinv)
    y = (xs[0] + xs[1] + xs[2] + xs[3]) * (0.1 / T)
    z = z_ref[...]
    for t in range(T):
        u = z + (xs[t] + y - z) * 0.5
        o = jnp.where(u > 1.0, 1.0, 0.0)
        z = u - o
        o_ref[t] = o
    znew_ref[...] = z


def _final_call(agg, q8, deg, z_seq):
    return pl.pallas_call(
        _final_body,
        grid=(N // NB,),
        in_specs=[
            pl.BlockSpec((NCH, NB, CW), lambda nb: (0, nb, 0)),
            pl.BlockSpec((NCH, NB, CW), lambda nb: (0, nb, 0)),
            pl.BlockSpec((NB, 1), lambda nb: (nb, 0)),
            pl.BlockSpec((NB, C), lambda nb: (nb, 0)),
        ],
        out_specs=[
            pl.BlockSpec((T, NB, C), lambda nb: (0, nb, 0)),
            pl.BlockSpec((NB, C), lambda nb: (nb, 0)),
        ],
        out_shape=[
            jax.ShapeDtypeStruct((T, N, C), jnp.float32),
            jax.ShapeDtypeStruct((N, C), jnp.float32),
        ],
    )(agg, q8, deg, z_seq)


def kernel(s_seq, z_seq, edge_index, W):
    ei = edge_index.astype(jnp.int32)
    src, dst = ei[0], ei[1]
    tiles_deg = NCORES * NTILES
    dst_deg = dst.reshape(NCORES, NTILES, E // (tiles_deg * EB), EB)
    dst_agg = dst.reshape(NTILES, E // (NTILES * EB), EB)
    srcoff = (src[None, :]
              + (jnp.arange(NCH, dtype=jnp.int32) * N)[:, None]
              ).reshape(NCH, NTILES, E // (NTILES * EB), EB)

    ones_c = jnp.ones((EB, CW), jnp.float32)
    zeros_w = jnp.zeros((32, CW), jnp.float32)

    part = _make_deg_kernel()(dst_deg, ones_c, zeros_w)
    q8, deg = _mm_scale_call(s_seq, part, W)
    q_flat = q8.reshape(NCH * N, CW)
    agg = _make_agg_kernel()(q_flat, srcoff, dst_agg, zeros_w)
    o_seq, z_new = _final_call(agg, q8, deg, z_seq)
    return (o_seq, z_new)
